# baseline (speedup 1.0000x reference)
import functools

import jax
import jax.numpy as jnp
from jax import lax
from jax.experimental import pallas as pl
from jax.experimental.pallas import tpu as pltpu
from jax.experimental.pallas import tpu_sc as plsc

_NC = 2
_NS = 16
_L = 16
_NW = _NC * _NS


@functools.lru_cache(maxsize=None)
def _build_sc_add(B, S, D, CH):
    seq_per_w = S // _NW
    n_chunks = seq_per_w // CH
    n_col = D // _L
    G = 8
    mesh = plsc.VectorSubcoreMesh(
        core_axis_name="c", subcore_axis_name="s",
        num_cores=_NC, num_subcores=_NS)

    @functools.partial(
        pl.kernel,
        out_type=jax.ShapeDtypeStruct((B, S, D), jnp.float32),
        mesh=mesh,
        scratch_types=[
            pltpu.VMEM((3, B, CH, D), jnp.float32),
            pltpu.VMEM((2, CH, D), jnp.float32),
            pltpu.SemaphoreType.DMA((3,)),
            pltpu.SemaphoreType.DMA((3,)),
            pltpu.SemaphoreType.DMA((2,)),
        ],
    )
    def run(x_hbm, pe_hbm, out_hbm, x_sl, pe_sl, in_sems, out_sems,
            pe_sems):
        wid = lax.axis_index("s") * _NC + lax.axis_index("c")
        s_base = wid * seq_per_w

        def start_pe(t, p):
            pltpu.async_copy(pe_hbm.at[pl.ds(s_base + t * CH, CH)],
                             pe_sl.at[p], pe_sems.at[p])

        def wait_pe(t, p):
            pltpu.make_async_copy(pe_hbm.at[pl.ds(s_base + t * CH, CH)],
                                  pe_sl.at[p], pe_sems.at[p]).wait()

        def start_in(t, p):
            pltpu.async_copy(x_hbm.at[:, pl.ds(s_base + t * CH, CH)],
                             x_sl.at[p], in_sems.at[p])

        def wait_in(t, p):
            pltpu.make_async_copy(x_hbm.at[:, pl.ds(s_base + t * CH, CH)],
                                  x_sl.at[p], in_sems.at[p]).wait()

        def start_out(t, p):
            pltpu.async_copy(x_sl.at[p],
                             out_hbm.at[:, pl.ds(s_base + t * CH, CH)],
                             out_sems.at[p])

        def wait_out(t, p):
            pltpu.make_async_copy(x_sl.at[p],
                                  out_hbm.at[:, pl.ds(s_base + t * CH, CH)],
                                  out_sems.at[p]).wait()

        start_pe(0, 0)
        start_in(0, 0)
        start_pe(1, 1)
        start_in(1, 1)

        def body(t, carry):
            p = lax.rem(t, 3)
            pp = lax.rem(t, 2)
            q = lax.rem(t + 2, 3)
            wait_pe(t, pp)
            wait_in(t, p)

            @plsc.parallel_loop(0, CH)
            def _(r):
                for g in range(n_col // G):
                    cols = [(g * G + j) * _L for j in range(G)]
                    pe_vs = [pe_sl[pp, r, pl.ds(c, _L)] for c in cols]
                    for b in range(B):
                        for c, pv in zip(cols, pe_vs):
                            plsc.addupdate(x_sl.at[p, b, r, pl.ds(c, _L)],
                                           pv)

            start_out(t, p)

            @pl.when(t < n_chunks - 2)
            def _():
                @pl.when(t > 0)
                def _():
                    wait_out(t - 1, q)
                start_in(t + 2, q)

            @pl.when(t < n_chunks - 2)
            def _():
                start_pe(t + 2, pp)
            return carry

        lax.fori_loop(0, n_chunks, body, 0)

        for t in range(n_chunks - 3, n_chunks):
            wait_out(t, t % 3)

    return run


def kernel(x, pe):
    B, S, D = x.shape
    run = _build_sc_add(B, S, D, 8)
    return run(x, pe)


# speedup vs baseline: 1.0156x; 1.0156x over previous
import functools

import jax
import jax.numpy as jnp
from jax import lax
from jax.experimental import pallas as pl
from jax.experimental.pallas import tpu as pltpu
from jax.experimental.pallas import tpu_sc as plsc

_NC = 2
_NS = 16
_L = 16
_NW = _NC * _NS


@functools.lru_cache(maxsize=None)
def _build_sc_add(B, S, D, CH):
    seq_per_w = S // _NW
    n_chunks = seq_per_w // CH
    n_col = D // _L
    G = 8
    mesh = plsc.VectorSubcoreMesh(
        core_axis_name="c", subcore_axis_name="s",
        num_cores=_NC, num_subcores=_NS)

    @functools.partial(
        pl.kernel,
        out_type=jax.ShapeDtypeStruct((B, S, D), jnp.float32),
        mesh=mesh,
        scratch_types=[
            pltpu.VMEM((6, B, CH, D), jnp.float32),
            pltpu.VMEM((2, CH, D), jnp.float32),
            pltpu.SemaphoreType.DMA((6,)),
            pltpu.SemaphoreType.DMA((6,)),
            pltpu.SemaphoreType.DMA((2,)),
        ],
    )
    def run(x_hbm, pe_hbm, out_hbm, x_sl, pe_sl, in_sems, out_sems,
            pe_sems):
        wid = lax.axis_index("s") * _NC + lax.axis_index("c")
        s_base = wid * seq_per_w

        def start_pe(t, p):
            pltpu.async_copy(pe_hbm.at[pl.ds(s_base + t * CH, CH)],
                             pe_sl.at[p], pe_sems.at[p])

        def wait_pe(t, p):
            pltpu.make_async_copy(pe_hbm.at[pl.ds(s_base + t * CH, CH)],
                                  pe_sl.at[p], pe_sems.at[p]).wait()

        def start_in(t, p):
            pltpu.async_copy(x_hbm.at[:, pl.ds(s_base + t * CH, CH)],
                             x_sl.at[p], in_sems.at[p])

        def wait_in(t, p):
            pltpu.make_async_copy(x_hbm.at[:, pl.ds(s_base + t * CH, CH)],
                                  x_sl.at[p], in_sems.at[p]).wait()

        def start_out(t, p):
            pltpu.async_copy(x_sl.at[p],
                             out_hbm.at[:, pl.ds(s_base + t * CH, CH)],
                             out_sems.at[p])

        def wait_out(t, p):
            pltpu.make_async_copy(x_sl.at[p],
                                  out_hbm.at[:, pl.ds(s_base + t * CH, CH)],
                                  out_sems.at[p]).wait()

        start_pe(0, 0)
        start_in(0, 0)
        start_pe(1, 1)
        start_in(1, 1)
        start_in(2, 2)
        start_in(3, 3)

        def body(t, carry):
            p = lax.rem(t, 6)
            pp = lax.rem(t, 2)
            q = lax.rem(t + 4, 6)
            wait_pe(t, pp)
            wait_in(t, p)

            @plsc.parallel_loop(0, CH)
            def _(r):
                for g in range(n_col // G):
                    cols = [(g * G + j) * _L for j in range(G)]
                    pe_vs = [pe_sl[pp, r, pl.ds(c, _L)] for c in cols]
                    for b in range(B):
                        for c, pv in zip(cols, pe_vs):
                            plsc.addupdate(x_sl.at[p, b, r, pl.ds(c, _L)],
                                           pv)

            start_out(t, p)

            @pl.when(t < n_chunks - 4)
            def _():
                @pl.when(t > 1)
                def _():
                    wait_out(t - 2, q)
                start_in(t + 4, q)

            @pl.when(t < n_chunks - 2)
            def _():
                start_pe(t + 2, pp)
            return carry

        lax.fori_loop(0, n_chunks, body, 0)

        for t in range(n_chunks - 6, n_chunks):
            wait_out(t, t % 6)

    return run


def kernel(x, pe):
    B, S, D = x.shape
    run = _build_sc_add(B, S, D, 4)
    return run(x, pe)


# speedup vs baseline: 1.0416x; 1.0256x over previous
import functools

import jax
import jax.numpy as jnp
from jax import lax
from jax.experimental import pallas as pl
from jax.experimental.pallas import tpu as pltpu
from jax.experimental.pallas import tpu_sc as plsc

_NC = 2
_NS = 16
_L = 16
_NW = _NC * _NS


@functools.lru_cache(maxsize=None)
def _build_sc_add(B, S, D, CH):
    seq_per_w = S // _NW
    n_chunks = seq_per_w // CH
    n_col = D // _L
    G = 8
    mesh = plsc.VectorSubcoreMesh(
        core_axis_name="c", subcore_axis_name="s",
        num_cores=_NC, num_subcores=_NS)

    @functools.partial(
        pl.kernel,
        out_type=jax.ShapeDtypeStruct((B, S, D), jnp.float32),
        mesh=mesh,
        scratch_types=[
            pltpu.VMEM((12, B, CH, D), jnp.float32),
            pltpu.VMEM((4, CH, D), jnp.float32),
            pltpu.SemaphoreType.DMA((12,)),
            pltpu.SemaphoreType.DMA((12,)),
            pltpu.SemaphoreType.DMA((4,)),
        ],
    )
    def run(x_hbm, pe_hbm, out_hbm, x_sl, pe_sl, in_sems, out_sems,
            pe_sems):
        wid = lax.axis_index("s") * _NC + lax.axis_index("c")
        s_base = wid * seq_per_w

        def start_pe(t, p):
            pltpu.async_copy(pe_hbm.at[pl.ds(s_base + t * CH, CH)],
                             pe_sl.at[p], pe_sems.at[p])

        def wait_pe(t, p):
            pltpu.make_async_copy(pe_hbm.at[pl.ds(s_base + t * CH, CH)],
                                  pe_sl.at[p], pe_sems.at[p]).wait()

        def start_in(t, p):
            pltpu.async_copy(x_hbm.at[:, pl.ds(s_base + t * CH, CH)],
                             x_sl.at[p], in_sems.at[p])

        def wait_in(t, p):
            pltpu.make_async_copy(x_hbm.at[:, pl.ds(s_base + t * CH, CH)],
                                  x_sl.at[p], in_sems.at[p]).wait()

        def start_out(t, p):
            pltpu.async_copy(x_sl.at[p],
                             out_hbm.at[:, pl.ds(s_base + t * CH, CH)],
                             out_sems.at[p])

        def wait_out(t, p):
            pltpu.make_async_copy(x_sl.at[p],
                                  out_hbm.at[:, pl.ds(s_base + t * CH, CH)],
                                  out_sems.at[p]).wait()

        for t0 in range(4):
            start_pe(t0, t0)
            start_in(t0, t0)
        for t0 in range(4, 8):
            start_in(t0, t0)

        def body(t, carry):
            p = lax.rem(t, 12)
            pp = lax.rem(t, 4)
            q = lax.rem(t + 8, 12)
            wait_pe(t, pp)
            wait_in(t, p)

            @plsc.parallel_loop(0, CH)
            def _(r):
                for g in range(n_col // G):
                    cols = [(g * G + j) * _L for j in range(G)]
                    pe_vs = [pe_sl[pp, r, pl.ds(c, _L)] for c in cols]
                    for b in range(B):
                        for c, pv in zip(cols, pe_vs):
                            plsc.addupdate(x_sl.at[p, b, r, pl.ds(c, _L)],
                                           pv)

            start_out(t, p)

            @pl.when(t < n_chunks - 8)
            def _():
                @pl.when(t > 3)
                def _():
                    wait_out(t - 4, q)
                start_in(t + 8, q)

            @pl.when(t < n_chunks - 4)
            def _():
                start_pe(t + 4, pp)
            return carry

        lax.fori_loop(0, n_chunks, body, 0)

        for t in range(n_chunks - 12, n_chunks):
            wait_out(t, t % 12)

    return run


def kernel(x, pe):
    B, S, D = x.shape
    run = _build_sc_add(B, S, D, 2)
    return run(x, pe)
